# Initial kernel scaffold; baseline (speedup 1.0000x reference)
import jax
import jax.numpy as jnp
from jax.experimental import pallas as pl


def kernel(x, edge_index, W_gcn, b_gcn, fc_W, fc_b):
    raise NotImplementedError("write your pallas kernel here")


# speedup vs baseline: 28.5073x; 28.5073x over previous
import functools

import jax
import jax.numpy as jnp
from jax import lax
from jax.experimental import pallas as pl
from jax.experimental.pallas import tpu as pltpu
from jax.experimental.pallas import tpu_sc as plsc

NFEAT = 128
D = 48
CH = 128
NC = 2
NS = 16
NW = NC * NS
N_PAD = 10240
RPT = N_PAD // NS
DEGW = 16


def _sc_degree(dst_w):
    per_w = dst_w.shape[1]
    mesh = plsc.VectorSubcoreMesh(core_axis_name="c", subcore_axis_name="s")

    @functools.partial(
        pl.kernel,
        mesh=mesh,
        out_type=jax.ShapeDtypeStruct((NC, N_PAD, DEGW), jnp.float32),
        compiler_params=pltpu.CompilerParams(use_tc_tiling_on_sc=False),
        scratch_types=[
            pltpu.VMEM((per_w, CH), jnp.int32),
            pltpu.VMEM((CH, DEGW), jnp.float32),
            pltpu.VMEM((64, DEGW), jnp.float32),
            pltpu.VMEM_SHARED((N_PAD, DEGW), jnp.float32),
        ],
    )
    def k(dst_hbm, out_hbm, idx_v, ones_v, zbuf, acc):
        c = lax.axis_index("c")
        s = lax.axis_index("s")
        wid = c * NS + s

        def fill_ones(r, carry):
            ones_v[r] = jnp.ones((16,), jnp.float32)
            return carry

        lax.fori_loop(0, CH, fill_ones, 0)

        def fill_zero(r, carry):
            zbuf[r] = jnp.zeros((16,), jnp.float32)
            return carry

        lax.fori_loop(0, 64, fill_zero, 0)

        base = s * RPT

        def zero_slab(i, carry):
            pltpu.sync_copy(zbuf, acc.at[pl.ds(base + i * 64, 64)])
            return carry

        lax.fori_loop(0, RPT // 64, zero_slab, 0)

        pltpu.sync_copy(dst_hbm.at[wid], idx_v)
        plsc.subcore_barrier()

        def body(j, carry):
            pltpu.sync_copy(ones_v, acc.at[idx_v.at[j]], add=True)
            return carry

        lax.fori_loop(0, per_w, body, 0)
        plsc.subcore_barrier()

        pltpu.sync_copy(acc.at[pl.ds(base, RPT)],
                        out_hbm.at[c, pl.ds(base, RPT)])

    return k(dst_w)


def _sc_edges(src_w, dst_w, h2):
    per_w = src_w.shape[1]
    mesh = plsc.VectorSubcoreMesh(core_axis_name="c", subcore_axis_name="s")

    @functools.partial(
        pl.kernel,
        mesh=mesh,
        out_type=jax.ShapeDtypeStruct((NC, N_PAD, D), jnp.float32),
        compiler_params=pltpu.CompilerParams(use_tc_tiling_on_sc=False),
        scratch_types=[
            pltpu.VMEM((per_w, CH), jnp.int32),
            pltpu.VMEM((per_w, CH), jnp.int32),
            pltpu.VMEM((CH, D), jnp.float32),
            pltpu.VMEM((64, D), jnp.float32),
            pltpu.VMEM_SHARED((N_PAD, D), jnp.float32),
            pltpu.SemaphoreType.DMA,
        ],
    )
    def k(src_hbm, dst_hbm, h2_hbm, out_hbm, src_v, dst_v, rows_v, zbuf, acc, sem):
        c = lax.axis_index("c")
        s = lax.axis_index("s")
        wid = c * NS + s

        def fill_zero(r, carry):
            for col in range(D // 16):
                zbuf[r, pl.ds(col * 16, 16)] = jnp.zeros((16,), jnp.float32)
            return carry

        lax.fori_loop(0, 64, fill_zero, 0)

        base = s * RPT

        def zero_slab(i, carry):
            pltpu.sync_copy(zbuf, acc.at[pl.ds(base + i * 64, 64)])
            return carry

        lax.fori_loop(0, RPT // 64, zero_slab, 0)

        pltpu.sync_copy(src_hbm.at[wid], src_v)
        pltpu.sync_copy(dst_hbm.at[wid], dst_v)
        plsc.subcore_barrier()

        def body(j, carry):
            pltpu.async_copy(h2_hbm.at[src_v.at[j]], rows_v, sem).wait()
            pltpu.sync_copy(rows_v, acc.at[dst_v.at[j]], add=True)
            return carry

        lax.fori_loop(0, per_w, body, 0)
        plsc.subcore_barrier()

        pltpu.sync_copy(acc.at[pl.ds(base, RPT)],
                        out_hbm.at[c, pl.ds(base, RPT)])

    return k(src_w, dst_w, h2)


def _tc_prepare(x_pad, W_gcn, fcWp, degp):
    def body(x_ref, w_ref, f_ref, deg_ref, h2_ref):
        wc = lax.dot_general(w_ref[:], f_ref[:], (((1,), (1,)), ((), ())),
                             preferred_element_type=jnp.float32)
        deg = deg_ref[0][:, :1] + deg_ref[1][:, :1] + 1.0
        dinv = lax.rsqrt(deg)
        g = lax.dot_general(x_ref[:], wc, (((1,), (0,)), ((), ())),
                            preferred_element_type=jnp.float32)
        h2_ref[:] = g * dinv

    return pl.pallas_call(
        body,
        out_shape=jax.ShapeDtypeStruct((N_PAD, D), jnp.float32),
    )(x_pad, W_gcn, fcWp, degp)


def _tc_finalize(partials, h2, degp, b2, fcWp, fcb2):
    def body(p_ref, h2_ref, deg_ref, b_ref, f_ref, fb_ref, y_ref):
        deg = deg_ref[0][:, :1] + deg_ref[1][:, :1] + 1.0
        dinv = lax.rsqrt(deg)
        tot = p_ref[0] + p_ref[1] + h2_ref[:]
        const = lax.dot_general(b_ref[:], f_ref[:], (((1,), (1,)), ((), ())),
                                preferred_element_type=jnp.float32) + fb_ref[:]
        y_ref[:] = tot * dinv + const

    return pl.pallas_call(
        body,
        out_shape=jax.ShapeDtypeStruct((N_PAD, D), jnp.float32),
    )(partials, h2, degp, b2, fcWp, fcb2)


def kernel(x, edge_index, W_gcn, b_gcn, fc_W, fc_b):
    N = x.shape[0]
    nclass = fc_W.shape[0]
    src = edge_index[0].astype(jnp.int32)
    dst = edge_index[1].astype(jnp.int32)
    E = src.shape[0]
    per_w = -(-E // (NW * CH))
    e_pad = NW * per_w * CH
    pad_idx = jnp.full((e_pad - E,), N, jnp.int32)
    src_w = jnp.concatenate([src, pad_idx]).reshape(NW, per_w, CH)
    dst_w = jnp.concatenate([dst, pad_idx]).reshape(NW, per_w, CH)
    x_pad = jnp.zeros((N_PAD, NFEAT), x.dtype).at[:N].set(x)
    fcWp = jnp.zeros((D, NFEAT), fc_W.dtype).at[:nclass].set(fc_W)
    fcb2 = jnp.zeros((1, D), fc_b.dtype).at[0, :nclass].set(fc_b)
    b2 = b_gcn.reshape(1, NFEAT)

    degp = _sc_degree(dst_w)
    h2 = _tc_prepare(x_pad, W_gcn, fcWp, degp)
    partials = _sc_edges(src_w, dst_w, h2)
    y48 = _tc_finalize(partials, h2, degp, b2, fcWp, fcb2)
    return y48[:N, :nclass]


# speedup vs baseline: 31.3881x; 1.1011x over previous
import functools

import jax
import jax.numpy as jnp
from jax import lax
from jax.experimental import pallas as pl
from jax.experimental.pallas import tpu as pltpu
from jax.experimental.pallas import tpu_sc as plsc

NFEAT = 128
D = 48
CH = 128
NC = 2
NS = 16
NW = NC * NS
N_PAD = 10240
RPT = N_PAD // NS
DEGW = 16
NBUF = 8


def _sc_degree(dst_w):
    per_w = dst_w.shape[1]
    mesh = plsc.VectorSubcoreMesh(core_axis_name="c", subcore_axis_name="s")

    @functools.partial(
        pl.kernel,
        mesh=mesh,
        out_type=jax.ShapeDtypeStruct((NC, N_PAD, DEGW), jnp.float32),
        compiler_params=pltpu.CompilerParams(use_tc_tiling_on_sc=False),
        scratch_types=[
            pltpu.VMEM((per_w, CH), jnp.int32),
            pltpu.VMEM((CH, DEGW), jnp.float32),
            pltpu.VMEM((64, DEGW), jnp.float32),
            pltpu.VMEM_SHARED((N_PAD, DEGW), jnp.float32),
            pltpu.SemaphoreType.DMA,
        ],
    )
    def k(dst_hbm, out_hbm, idx_v, ones_v, zbuf, acc, dsem):
        c = lax.axis_index("c")
        s = lax.axis_index("s")
        wid = c * NS + s

        def fill_ones(r, carry):
            ones_v[r] = jnp.ones((16,), jnp.float32)
            return carry

        lax.fori_loop(0, CH, fill_ones, 0)

        def fill_zero(r, carry):
            zbuf[r] = jnp.zeros((16,), jnp.float32)
            return carry

        lax.fori_loop(0, 64, fill_zero, 0)

        base = s * RPT

        def zero_slab(i, carry):
            pltpu.sync_copy(zbuf, acc.at[pl.ds(base + i * 64, 64)])
            return carry

        lax.fori_loop(0, RPT // 64, zero_slab, 0)

        pltpu.sync_copy(dst_hbm.at[wid], idx_v)
        plsc.subcore_barrier()

        def body(j, carry):
            pltpu.async_copy(ones_v, acc.at[idx_v.at[j]], dsem, add=True)
            return carry

        lax.fori_loop(0, per_w, body, 0)

        def drain(j, carry):
            pltpu.make_async_copy(ones_v, acc.at[idx_v.at[j]], dsem).wait()
            return carry

        lax.fori_loop(0, per_w, drain, 0)
        plsc.subcore_barrier()

        pltpu.sync_copy(acc.at[pl.ds(base, RPT)],
                        out_hbm.at[c, pl.ds(base, RPT)])

    return k(dst_w)


def _sc_edges(src_w, dst_w, h2):
    per_w = src_w.shape[1]
    mesh = plsc.VectorSubcoreMesh(core_axis_name="c", subcore_axis_name="s")

    @functools.partial(
        pl.kernel,
        mesh=mesh,
        out_type=jax.ShapeDtypeStruct((NC, N_PAD, D), jnp.float32),
        compiler_params=pltpu.CompilerParams(use_tc_tiling_on_sc=False),
        scratch_types=[
            pltpu.VMEM((per_w, CH), jnp.int32),
            pltpu.VMEM((per_w, CH), jnp.int32),
            pltpu.VMEM((NBUF, CH, D), jnp.float32),
            pltpu.VMEM((64, D), jnp.float32),
            pltpu.VMEM_SHARED((N_PAD, D), jnp.float32),
            pltpu.SemaphoreType.DMA((NBUF,)),
            pltpu.SemaphoreType.DMA((NBUF,)),
        ],
    )
    def k(src_hbm, dst_hbm, h2_hbm, out_hbm, src_v, dst_v, rows, zbuf, acc,
          gsem, ssem):
        c = lax.axis_index("c")
        s = lax.axis_index("s")
        wid = c * NS + s

        def fill_zero(r, carry):
            for col in range(D // 16):
                zbuf[r, pl.ds(col * 16, 16)] = jnp.zeros((16,), jnp.float32)
            return carry

        lax.fori_loop(0, 64, fill_zero, 0)

        base = s * RPT

        def zero_slab(i, carry):
            pltpu.sync_copy(zbuf, acc.at[pl.ds(base + i * 64, 64)])
            return carry

        lax.fori_loop(0, RPT // 64, zero_slab, 0)

        pltpu.sync_copy(src_hbm.at[wid], src_v)
        pltpu.sync_copy(dst_hbm.at[wid], dst_v)
        plsc.subcore_barrier()

        rounds = per_w // NBUF
        for b in range(NBUF):
            pltpu.async_copy(h2_hbm.at[src_v.at[b]], rows.at[b], gsem.at[b])

        def round_body(r, carry):
            for b in range(NBUF):
                j = r * NBUF + b
                pltpu.make_async_copy(h2_hbm.at[src_v.at[j]], rows.at[b],
                                      gsem.at[b]).wait()
                pltpu.async_copy(rows.at[b], acc.at[dst_v.at[j]], ssem.at[b],
                                 add=True)

            @pl.when(r < rounds - 1)
            def _issue_next():
                for b in range(NBUF):
                    jn = (r + 1) * NBUF + b
                    pltpu.make_async_copy(rows.at[b], acc.at[dst_v.at[jn]],
                                          ssem.at[b]).wait()
                    pltpu.async_copy(h2_hbm.at[src_v.at[jn]], rows.at[b],
                                     gsem.at[b])

            return carry

        lax.fori_loop(0, rounds, round_body, 0)
        for b in range(NBUF):
            pltpu.make_async_copy(rows.at[b], acc.at[dst_v.at[b]],
                                  ssem.at[b]).wait()
        plsc.subcore_barrier()

        pltpu.sync_copy(acc.at[pl.ds(base, RPT)],
                        out_hbm.at[c, pl.ds(base, RPT)])

    return k(src_w, dst_w, h2)


def _tc_prepare(x_pad, W_gcn, fcWp, degp):
    def body(x_ref, w_ref, f_ref, deg_ref, h2_ref):
        wc = lax.dot_general(w_ref[:], f_ref[:], (((1,), (1,)), ((), ())),
                             preferred_element_type=jnp.float32)
        deg = deg_ref[0][:, :1] + deg_ref[1][:, :1] + 1.0
        dinv = lax.rsqrt(deg)
        g = lax.dot_general(x_ref[:], wc, (((1,), (0,)), ((), ())),
                            preferred_element_type=jnp.float32)
        h2_ref[:] = g * dinv

    return pl.pallas_call(
        body,
        out_shape=jax.ShapeDtypeStruct((N_PAD, D), jnp.float32),
    )(x_pad, W_gcn, fcWp, degp)


def _tc_finalize(partials, h2, degp, b2, fcWp, fcb2):
    def body(p_ref, h2_ref, deg_ref, b_ref, f_ref, fb_ref, y_ref):
        deg = deg_ref[0][:, :1] + deg_ref[1][:, :1] + 1.0
        dinv = lax.rsqrt(deg)
        tot = p_ref[0] + p_ref[1] + h2_ref[:]
        const = lax.dot_general(b_ref[:], f_ref[:], (((1,), (1,)), ((), ())),
                                preferred_element_type=jnp.float32) + fb_ref[:]
        y_ref[:] = tot * dinv + const

    return pl.pallas_call(
        body,
        out_shape=jax.ShapeDtypeStruct((N_PAD, D), jnp.float32),
    )(partials, h2, degp, b2, fcWp, fcb2)


def kernel(x, edge_index, W_gcn, b_gcn, fc_W, fc_b):
    N = x.shape[0]
    nclass = fc_W.shape[0]
    src = edge_index[0].astype(jnp.int32)
    dst = edge_index[1].astype(jnp.int32)
    E = src.shape[0]
    per_w = -(-E // (NW * CH))
    per_w = -(-per_w // NBUF) * NBUF
    e_pad = NW * per_w * CH
    pad_idx = jnp.full((e_pad - E,), N, jnp.int32)
    src_w = jnp.concatenate([src, pad_idx]).reshape(NW, per_w, CH)
    dst_w = jnp.concatenate([dst, pad_idx]).reshape(NW, per_w, CH)
    x_pad = jnp.zeros((N_PAD, NFEAT), x.dtype).at[:N].set(x)
    fcWp = jnp.zeros((D, NFEAT), fc_W.dtype).at[:nclass].set(fc_W)
    fcb2 = jnp.zeros((1, D), fc_b.dtype).at[0, :nclass].set(fc_b)
    b2 = b_gcn.reshape(1, NFEAT)

    degp = _sc_degree(dst_w)
    h2 = _tc_prepare(x_pad, W_gcn, fcWp, degp)
    partials = _sc_edges(src_w, dst_w, h2)
    y48 = _tc_finalize(partials, h2, degp, b2, fcWp, fcb2)
    return y48[:N, :nclass]
